# initial kernel scaffold (unmeasured)
import jax
import jax.numpy as jnp
from jax import lax
from jax.experimental import pallas as pl
from jax.experimental.pallas import tpu as pltpu


def kernel(
    x,
):
    def body(*refs):
        pass

    out_shape = jax.ShapeDtypeStruct(..., jnp.float32)
    return pl.pallas_call(body, out_shape=out_shape)(...)


# baseline (device time: 827989 ns/iter reference)
import jax
import jax.numpy as jnp
from jax import lax
from jax.experimental import pallas as pl
from jax.experimental.pallas import tpu as pltpu

M, N = 4096, 2048
CHUNK = 1024
N_CHUNKS = M // CHUNK


def kernel(x):
    x = x.reshape(M, N)

    def body(x_ref, out_ref, rx_ref, ry_ref, va, vb,

             send_sem_x, recv_sem_x, send_sem_y, recv_sem_y,
             cp_sem_a, cp_sem_b):
        my_x = lax.axis_index("x")
        my_y = lax.axis_index("y")
        nbr_x = (1 - my_x, my_y)
        nbr_y = (my_x, 1 - my_y)

        barrier = pltpu.get_barrier_semaphore()
        pl.semaphore_signal(barrier, inc=1, device_id=nbr_x,
                            device_id_type=pl.DeviceIdType.MESH)
        pl.semaphore_signal(barrier, inc=1, device_id=nbr_y,
                            device_id_type=pl.DeviceIdType.MESH)
        pl.semaphore_wait(barrier, 2)

        rdma_x = pltpu.make_async_remote_copy(
            src_ref=x_ref, dst_ref=rx_ref,
            send_sem=send_sem_x, recv_sem=recv_sem_x,
            device_id=nbr_x, device_id_type=pl.DeviceIdType.MESH,
        )
        rdma_x.start()
        rdma_x.wait()

        for k in range(N_CHUNKS):
            sl = pl.ds(k * CHUNK, CHUNK)
            cp_in_a = pltpu.make_async_copy(x_ref.at[sl], va, cp_sem_a)
            cp_in_b = pltpu.make_async_copy(rx_ref.at[sl], vb, cp_sem_b)
            cp_in_a.start()
            cp_in_b.start()
            cp_in_a.wait()
            cp_in_b.wait()
            va[...] = va[...] + vb[...]
            cp_out = pltpu.make_async_copy(va, out_ref.at[sl], cp_sem_a)
            cp_out.start()
            cp_out.wait()

        rdma_y = pltpu.make_async_remote_copy(
            src_ref=out_ref, dst_ref=ry_ref,
            send_sem=send_sem_y, recv_sem=recv_sem_y,
            device_id=nbr_y, device_id_type=pl.DeviceIdType.MESH,
        )
        rdma_y.start()
        rdma_y.wait()

        for k in range(N_CHUNKS):
            sl = pl.ds(k * CHUNK, CHUNK)
            cp_in_a = pltpu.make_async_copy(out_ref.at[sl], va, cp_sem_a)
            cp_in_b = pltpu.make_async_copy(ry_ref.at[sl], vb, cp_sem_b)
            cp_in_a.start()
            cp_in_b.start()
            cp_in_a.wait()
            cp_in_b.wait()
            va[...] = va[...] + vb[...]
            cp_out = pltpu.make_async_copy(va, out_ref.at[sl], cp_sem_a)
            cp_out.start()
            cp_out.wait()

    out, _, _ = pl.pallas_call(
        body,
        out_shape=(
            jax.ShapeDtypeStruct((M, N), jnp.float32),
            jax.ShapeDtypeStruct((M, N), jnp.float32),
            jax.ShapeDtypeStruct((M, N), jnp.float32),
        ),
        in_specs=[pl.BlockSpec(memory_space=pltpu.MemorySpace.HBM)],
        out_specs=(
            pl.BlockSpec(memory_space=pltpu.MemorySpace.HBM),
            pl.BlockSpec(memory_space=pltpu.MemorySpace.HBM),
            pl.BlockSpec(memory_space=pltpu.MemorySpace.HBM),
        ),
        scratch_shapes=[
            pltpu.VMEM((CHUNK, N), jnp.float32),
            pltpu.VMEM((CHUNK, N), jnp.float32),
            pltpu.SemaphoreType.DMA,
            pltpu.SemaphoreType.DMA,
            pltpu.SemaphoreType.DMA,
            pltpu.SemaphoreType.DMA,
            pltpu.SemaphoreType.DMA,
            pltpu.SemaphoreType.DMA,
        ],
        compiler_params=pltpu.CompilerParams(collective_id=0),
    )(x)
    return out


# device time: 330096 ns/iter; 2.5083x vs baseline; 2.5083x over previous
import jax
import jax.numpy as jnp
from jax import lax
from jax.experimental import pallas as pl
from jax.experimental.pallas import tpu as pltpu

M, N = 4096, 2048
Q = 1024
E = 512


def kernel(x):
    x = x.reshape(M, N)

    def body(x_ref, out_ref, rA2_ref, rB2_ref, va, vb,
             sendA, recvA, sendB, recvB, cp_a, cp_b):
        my_x = lax.axis_index("x")
        my_y = lax.axis_index("y")
        nbr_x = (1 - my_x, my_y)
        nbr_y = (my_x, 1 - my_y)

        qA_keep = my_x * Q
        qA_send = (1 - my_x) * Q
        qB_keep = 2 * Q + my_y * Q
        qB_send = 2 * Q + (1 - my_y) * Q
        eA_keep = my_x * Q + my_y * E
        eA_send = my_x * Q + (1 - my_y) * E
        eB_keep = 2 * Q + my_y * Q + my_x * E
        eB_send = 2 * Q + my_y * Q + (1 - my_x) * E

        def rdma(src, dst, ssem, rsem, dev):
            return pltpu.make_async_remote_copy(
                src_ref=src, dst_ref=dst, send_sem=ssem, recv_sem=rsem,
                device_id=dev, device_id_type=pl.DeviceIdType.MESH,
            )

        def add_into_out(dst_start, rows, a_ref, a_start, b_ref, b_start):
            va_s = va.at[pl.ds(0, rows)]
            vb_s = vb.at[pl.ds(0, rows)]
            cp1 = pltpu.make_async_copy(a_ref.at[pl.ds(a_start, rows)], va_s, cp_a)
            cp2 = pltpu.make_async_copy(b_ref.at[pl.ds(b_start, rows)], vb_s, cp_b)
            cp1.start()
            cp2.start()
            cp1.wait()
            cp2.wait()
            va[pl.ds(0, rows), :] = va[pl.ds(0, rows), :] + vb[pl.ds(0, rows), :]
            cp3 = pltpu.make_async_copy(va_s, out_ref.at[pl.ds(dst_start, rows)], cp_a)
            cp3.start()
            cp3.wait()

        barrier = pltpu.get_barrier_semaphore()
        pl.semaphore_signal(barrier, inc=1, device_id=nbr_x,
                            device_id_type=pl.DeviceIdType.MESH)
        pl.semaphore_signal(barrier, inc=1, device_id=nbr_y,
                            device_id_type=pl.DeviceIdType.MESH)
        pl.semaphore_wait(barrier, 2)

        a1 = rdma(x_ref.at[pl.ds(qA_send, Q)], out_ref.at[pl.ds(qA_send, Q)],
                  sendA.at[0], recvA.at[0], nbr_x)
        b1 = rdma(x_ref.at[pl.ds(qB_send, Q)], out_ref.at[pl.ds(qB_send, Q)],
                  sendB.at[0], recvB.at[0], nbr_y)
        a1.start()
        b1.start()

        a1.wait()
        add_into_out(qA_keep, Q, out_ref, qA_keep, x_ref, qA_keep)
        a2 = rdma(out_ref.at[pl.ds(eA_send, E)], rA2_ref,
                  sendA.at[1], recvA.at[1], nbr_y)
        a2.start()

        b1.wait()
        add_into_out(qB_keep, Q, out_ref, qB_keep, x_ref, qB_keep)
        b2 = rdma(out_ref.at[pl.ds(eB_send, E)], rB2_ref,
                  sendB.at[1], recvB.at[1], nbr_x)
        b2.start()

        a2.wait()
        add_into_out(eA_keep, E, out_ref, eA_keep, rA2_ref, 0)
        a3 = rdma(out_ref.at[pl.ds(eA_keep, E)], out_ref.at[pl.ds(eA_keep, E)],
                  sendA.at[2], recvA.at[2], nbr_y)
        a3.start()

        b2.wait()
        add_into_out(eB_keep, E, out_ref, eB_keep, rB2_ref, 0)
        b3 = rdma(out_ref.at[pl.ds(eB_keep, E)], out_ref.at[pl.ds(eB_keep, E)],
                  sendB.at[2], recvB.at[2], nbr_x)
        b3.start()

        a3.wait()
        a4 = rdma(out_ref.at[pl.ds(qA_keep, Q)], out_ref.at[pl.ds(qA_keep, Q)],
                  sendA.at[3], recvA.at[3], nbr_x)
        a4.start()

        b3.wait()
        b4 = rdma(out_ref.at[pl.ds(qB_keep, Q)], out_ref.at[pl.ds(qB_keep, Q)],
                  sendB.at[3], recvB.at[3], nbr_y)
        b4.start()

        a4.wait()
        b4.wait()

    out, _, _ = pl.pallas_call(
        body,
        out_shape=(
            jax.ShapeDtypeStruct((M, N), jnp.float32),
            jax.ShapeDtypeStruct((E, N), jnp.float32),
            jax.ShapeDtypeStruct((E, N), jnp.float32),
        ),
        in_specs=[pl.BlockSpec(memory_space=pltpu.MemorySpace.HBM)],
        out_specs=(
            pl.BlockSpec(memory_space=pltpu.MemorySpace.HBM),
            pl.BlockSpec(memory_space=pltpu.MemorySpace.HBM),
            pl.BlockSpec(memory_space=pltpu.MemorySpace.HBM),
        ),
        scratch_shapes=[
            pltpu.VMEM((Q, N), jnp.float32),
            pltpu.VMEM((Q, N), jnp.float32),
            pltpu.SemaphoreType.DMA((4,)),
            pltpu.SemaphoreType.DMA((4,)),
            pltpu.SemaphoreType.DMA((4,)),
            pltpu.SemaphoreType.DMA((4,)),
            pltpu.SemaphoreType.DMA,
            pltpu.SemaphoreType.DMA,
        ],
        compiler_params=pltpu.CompilerParams(collective_id=0),
    )(x)
    return out


# device time: 308789 ns/iter; 2.6814x vs baseline; 1.0690x over previous
import jax
import jax.numpy as jnp
from jax import lax
from jax.experimental import pallas as pl
from jax.experimental.pallas import tpu as pltpu

M, N = 4096, 2048
Q = 1024
E = 512

P1A, P1B, P2, P3, P4A, P4B = range(6)


def kernel(x):
    x = x.reshape(M, N)

    def body(x_ref, out_ref, rA2_ref, rB2_ref, va, vb, vc, vd,
             sendA, recvA, sendB, recvB, cpa1, cpa2, cpb1, cpb2):
        my_x = lax.axis_index("x")
        my_y = lax.axis_index("y")
        nbr_x = (1 - my_x, my_y)
        nbr_y = (my_x, 1 - my_y)

        qA_keep = my_x * Q
        qA_send = (1 - my_x) * Q
        qB_keep = 2 * Q + my_y * Q
        qB_send = 2 * Q + (1 - my_y) * Q
        eA_keep = qA_keep + my_y * E
        eA_send = qA_keep + (1 - my_y) * E
        eB_keep = qB_keep + my_x * E
        eB_send = qB_keep + (1 - my_x) * E
        p1A_first = qA_send + (1 - my_y) * E
        p1A_second = qA_send + my_y * E
        p1B_first = qB_send + (1 - my_x) * E
        p1B_second = qB_send + my_x * E

        def remote(src_start, dst_start, rows, ssem, rsem, dev,
                   src_ref=None, dst_ref=None):
            src = (out_ref if src_ref is None else src_ref).at[pl.ds(src_start, rows)]
            dst = out_ref.at[pl.ds(dst_start, rows)] if dst_ref is None else dst_ref
            return pltpu.make_async_remote_copy(
                src_ref=src, dst_ref=dst, send_sem=ssem, recv_sem=rsem,
                device_id=dev, device_id_type=pl.DeviceIdType.MESH,
            )

        def add_into_out(dst_start, a_ref, a_start, b_ref, b_start, v1, v2, s1, s2):
            cp1 = pltpu.make_async_copy(a_ref.at[pl.ds(a_start, E)], v1, s1)
            cp2 = pltpu.make_async_copy(b_ref.at[pl.ds(b_start, E)], v2, s2)
            cp1.start()
            cp2.start()
            cp1.wait()
            cp2.wait()
            v1[...] = v1[...] + v2[...]
            cp3 = pltpu.make_async_copy(v1, out_ref.at[pl.ds(dst_start, E)], s1)
            cp3.start()
            cp3.wait()

        barrier = pltpu.get_barrier_semaphore()
        pl.semaphore_signal(barrier, inc=1, device_id=nbr_x,
                            device_id_type=pl.DeviceIdType.MESH)
        pl.semaphore_signal(barrier, inc=1, device_id=nbr_y,
                            device_id_type=pl.DeviceIdType.MESH)
        pl.semaphore_wait(barrier, 2)

        a1a = remote(p1A_first, p1A_first, E, sendA.at[P1A], recvA.at[P1A],
                     nbr_x, src_ref=x_ref)
        a1b = remote(p1A_second, p1A_second, E, sendA.at[P1B], recvA.at[P1B],
                     nbr_x, src_ref=x_ref)
        b1a = remote(p1B_first, p1B_first, E, sendB.at[P1A], recvB.at[P1A],
                     nbr_y, src_ref=x_ref)
        b1b = remote(p1B_second, p1B_second, E, sendB.at[P1B], recvB.at[P1B],
                     nbr_y, src_ref=x_ref)
        a1a.start()
        b1a.start()
        a1b.start()
        b1b.start()

        a1a.wait()
        add_into_out(eA_send, out_ref, eA_send, x_ref, eA_send, va, vb, cpa1, cpa2)
        a2 = remote(eA_send, 0, E, sendA.at[P2], recvA.at[P2], nbr_y,
                    dst_ref=rA2_ref)
        a2.start()

        b1a.wait()
        add_into_out(eB_send, out_ref, eB_send, x_ref, eB_send, vc, vd, cpb1, cpb2)
        b2 = remote(eB_send, 0, E, sendB.at[P2], recvB.at[P2], nbr_x,
                    dst_ref=rB2_ref)
        b2.start()

        a1b.wait()
        add_into_out(eA_keep, out_ref, eA_keep, x_ref, eA_keep, va, vb, cpa1, cpa2)
        b1b.wait()
        add_into_out(eB_keep, out_ref, eB_keep, x_ref, eB_keep, vc, vd, cpb1, cpb2)

        a2.wait()
        add_into_out(eA_keep, out_ref, eA_keep, rA2_ref, 0, va, vb, cpa1, cpa2)
        a3 = remote(eA_keep, eA_keep, E, sendA.at[P3], recvA.at[P3], nbr_y)
        a3.start()
        a4a = remote(eA_keep, eA_keep, E, sendA.at[P4A], recvA.at[P4A], nbr_x)
        a4a.start()

        b2.wait()
        add_into_out(eB_keep, out_ref, eB_keep, rB2_ref, 0, vc, vd, cpb1, cpb2)
        b3 = remote(eB_keep, eB_keep, E, sendB.at[P3], recvB.at[P3], nbr_x)
        b3.start()
        b4a = remote(eB_keep, eB_keep, E, sendB.at[P4A], recvB.at[P4A], nbr_y)
        b4a.start()

        a3.wait()
        a4b = remote(eA_send, eA_send, E, sendA.at[P4B], recvA.at[P4B], nbr_x)
        a4b.start()
        b3.wait()
        b4b = remote(eB_send, eB_send, E, sendB.at[P4B], recvB.at[P4B], nbr_y)
        b4b.start()

        a4a.wait()
        b4a.wait()
        a4b.wait()
        b4b.wait()

    out, _, _ = pl.pallas_call(
        body,
        out_shape=(
            jax.ShapeDtypeStruct((M, N), jnp.float32),
            jax.ShapeDtypeStruct((E, N), jnp.float32),
            jax.ShapeDtypeStruct((E, N), jnp.float32),
        ),
        in_specs=[pl.BlockSpec(memory_space=pltpu.MemorySpace.HBM)],
        out_specs=(
            pl.BlockSpec(memory_space=pltpu.MemorySpace.HBM),
            pl.BlockSpec(memory_space=pltpu.MemorySpace.HBM),
            pl.BlockSpec(memory_space=pltpu.MemorySpace.HBM),
        ),
        scratch_shapes=[
            pltpu.VMEM((E, N), jnp.float32),
            pltpu.VMEM((E, N), jnp.float32),
            pltpu.VMEM((E, N), jnp.float32),
            pltpu.VMEM((E, N), jnp.float32),
            pltpu.SemaphoreType.DMA((6,)),
            pltpu.SemaphoreType.DMA((6,)),
            pltpu.SemaphoreType.DMA((6,)),
            pltpu.SemaphoreType.DMA((6,)),
            pltpu.SemaphoreType.DMA,
            pltpu.SemaphoreType.DMA,
            pltpu.SemaphoreType.DMA,
            pltpu.SemaphoreType.DMA,
        ],
        compiler_params=pltpu.CompilerParams(collective_id=0),
    )(x)
    return out


# device time: 302726 ns/iter; 2.7351x vs baseline; 1.0200x over previous
import jax
import jax.numpy as jnp
from jax import lax
from jax.experimental import pallas as pl
from jax.experimental.pallas import tpu as pltpu

M, N = 4096, 2048
Q = 1024
E = 512

P1A, P1B, P2, P3, P4A, P4B = range(6)


def kernel(x):
    x = x.reshape(M, N)

    def body(x_ref, out_ref, vrA, vrB, r2A, r2B, vxA_s, vxA_k, vxB_s, vxB_k,
             sendA, recvA, sendB, recvB, cpa, cpb, cpc, cpd):
        my_x = lax.axis_index("x")
        my_y = lax.axis_index("y")
        nbr_x = (1 - my_x, my_y)
        nbr_y = (my_x, 1 - my_y)

        qA_keep = my_x * Q
        qA_send = (1 - my_x) * Q
        qB_keep = 2 * Q + my_y * Q
        qB_send = 2 * Q + (1 - my_y) * Q
        eA_keep = qA_keep + my_y * E
        eA_send = qA_keep + (1 - my_y) * E
        eB_keep = qB_keep + my_x * E
        eB_send = qB_keep + (1 - my_x) * E
        p1A_first = qA_send + (1 - my_y) * E
        p1A_second = qA_send + my_y * E
        p1B_first = qB_send + (1 - my_x) * E
        p1B_second = qB_send + my_x * E

        def remote(src, dst, ssem, rsem, dev):
            return pltpu.make_async_remote_copy(
                src_ref=src, dst_ref=dst, send_sem=ssem, recv_sem=rsem,
                device_id=dev, device_id_type=pl.DeviceIdType.MESH,
            )

        pfs = []
        for buf, start, sem in (
            (vxA_s, eA_send, cpa), (vxB_s, eB_send, cpb),
            (vxA_k, eA_keep, cpc), (vxB_k, eB_keep, cpd),
        ):
            cp = pltpu.make_async_copy(x_ref.at[pl.ds(start, E)], buf, sem)
            cp.start()
            pfs.append(cp)

        barrier = pltpu.get_barrier_semaphore()
        pl.semaphore_signal(barrier, inc=1, device_id=nbr_x,
                            device_id_type=pl.DeviceIdType.MESH)
        pl.semaphore_signal(barrier, inc=1, device_id=nbr_y,
                            device_id_type=pl.DeviceIdType.MESH)
        pl.semaphore_wait(barrier, 2)

        a1a = remote(x_ref.at[pl.ds(p1A_first, E)], vrA.at[0],
                     sendA.at[P1A], recvA.at[P1A], nbr_x)
        a1b = remote(x_ref.at[pl.ds(p1A_second, E)], vrA.at[1],
                     sendA.at[P1B], recvA.at[P1B], nbr_x)
        b1a = remote(x_ref.at[pl.ds(p1B_first, E)], vrB.at[0],
                     sendB.at[P1A], recvB.at[P1A], nbr_y)
        b1b = remote(x_ref.at[pl.ds(p1B_second, E)], vrB.at[1],
                     sendB.at[P1B], recvB.at[P1B], nbr_y)
        a1a.start()
        b1a.start()
        a1b.start()
        b1b.start()

        a1a.wait()
        pfs[0].wait()
        vrA[0] = vrA[0] + vxA_s[...]
        a2 = remote(vrA.at[0], r2A, sendA.at[P2], recvA.at[P2], nbr_y)
        a2.start()

        b1a.wait()
        pfs[1].wait()
        vrB[0] = vrB[0] + vxB_s[...]
        b2 = remote(vrB.at[0], r2B, sendB.at[P2], recvB.at[P2], nbr_x)
        b2.start()

        a1b.wait()
        pfs[2].wait()
        vrA[1] = vrA[1] + vxA_k[...]
        b1b.wait()
        pfs[3].wait()
        vrB[1] = vrB[1] + vxB_k[...]

        a2.wait()
        vrA[1] = vrA[1] + r2A[...]
        a3 = remote(vrA.at[1], out_ref.at[pl.ds(eA_keep, E)],
                    sendA.at[P3], recvA.at[P3], nbr_y)
        a3.start()
        a4a = remote(vrA.at[1], out_ref.at[pl.ds(eA_keep, E)],
                     sendA.at[P4A], recvA.at[P4A], nbr_x)
        a4a.start()
        stA = pltpu.make_async_copy(vrA.at[1], out_ref.at[pl.ds(eA_keep, E)], cpa)
        stA.start()

        b2.wait()
        vrB[1] = vrB[1] + r2B[...]
        b3 = remote(vrB.at[1], out_ref.at[pl.ds(eB_keep, E)],
                    sendB.at[P3], recvB.at[P3], nbr_x)
        b3.start()
        b4a = remote(vrB.at[1], out_ref.at[pl.ds(eB_keep, E)],
                     sendB.at[P4A], recvB.at[P4A], nbr_y)
        b4a.start()
        stB = pltpu.make_async_copy(vrB.at[1], out_ref.at[pl.ds(eB_keep, E)], cpb)
        stB.start()

        a3.wait()
        a4b = remote(out_ref.at[pl.ds(eA_send, E)], out_ref.at[pl.ds(eA_send, E)],
                     sendA.at[P4B], recvA.at[P4B], nbr_x)
        a4b.start()
        b3.wait()
        b4b = remote(out_ref.at[pl.ds(eB_send, E)], out_ref.at[pl.ds(eB_send, E)],
                     sendB.at[P4B], recvB.at[P4B], nbr_y)
        b4b.start()

        stA.wait()
        stB.wait()
        a4a.wait()
        b4a.wait()
        a4b.wait()
        b4b.wait()

    out = pl.pallas_call(
        body,
        out_shape=jax.ShapeDtypeStruct((M, N), jnp.float32),
        in_specs=[pl.BlockSpec(memory_space=pltpu.MemorySpace.HBM)],
        out_specs=pl.BlockSpec(memory_space=pltpu.MemorySpace.HBM),
        scratch_shapes=[
            pltpu.VMEM((2, E, N), jnp.float32),
            pltpu.VMEM((2, E, N), jnp.float32),
            pltpu.VMEM((E, N), jnp.float32),
            pltpu.VMEM((E, N), jnp.float32),
            pltpu.VMEM((E, N), jnp.float32),
            pltpu.VMEM((E, N), jnp.float32),
            pltpu.VMEM((E, N), jnp.float32),
            pltpu.VMEM((E, N), jnp.float32),
            pltpu.SemaphoreType.DMA((6,)),
            pltpu.SemaphoreType.DMA((6,)),
            pltpu.SemaphoreType.DMA((6,)),
            pltpu.SemaphoreType.DMA((6,)),
            pltpu.SemaphoreType.DMA,
            pltpu.SemaphoreType.DMA,
            pltpu.SemaphoreType.DMA,
            pltpu.SemaphoreType.DMA,
        ],
        compiler_params=pltpu.CompilerParams(
            collective_id=0,
            vmem_limit_bytes=56 * 1024 * 1024,
        ),
    )(x)
    return out


# device time: 302637 ns/iter; 2.7359x vs baseline; 1.0003x over previous
import jax
import jax.numpy as jnp
from jax import lax
from jax.experimental import pallas as pl
from jax.experimental.pallas import tpu as pltpu

M, N = 4096, 2048
Q = 1024
E = 512

P1A, P1B, P2, P3, P4A, P4B = range(6)


def kernel(x):

    def body(x_ref, out_ref, vrA, vrB, r2A, r2B, vxA_s, vxA_k, vxB_s, vxB_k,
             sendA, recvA, sendB, recvB, cpa, cpb, cpc, cpd):
        my_x = lax.axis_index("x")
        my_y = lax.axis_index("y")
        nbr_x = (1 - my_x, my_y)
        nbr_y = (my_x, 1 - my_y)

        qA_keep = my_x * Q
        qA_send = (1 - my_x) * Q
        qB_keep = 2 * Q + my_y * Q
        qB_send = 2 * Q + (1 - my_y) * Q
        eA_keep = qA_keep + my_y * E
        eA_send = qA_keep + (1 - my_y) * E
        eB_keep = qB_keep + my_x * E
        eB_send = qB_keep + (1 - my_x) * E
        p1A_first = qA_send + (1 - my_y) * E
        p1A_second = qA_send + my_y * E
        p1B_first = qB_send + (1 - my_x) * E
        p1B_second = qB_send + my_x * E

        def remote(src, dst, ssem, rsem, dev):
            return pltpu.make_async_remote_copy(
                src_ref=src, dst_ref=dst, send_sem=ssem, recv_sem=rsem,
                device_id=dev, device_id_type=pl.DeviceIdType.MESH,
            )

        pfs = []
        for buf, start, sem in (
            (vxA_s, eA_send, cpa), (vxB_s, eB_send, cpb),
            (vxA_k, eA_keep, cpc), (vxB_k, eB_keep, cpd),
        ):
            cp = pltpu.make_async_copy(x_ref.at[0, 0, pl.ds(start, E)], buf, sem)
            cp.start()
            pfs.append(cp)

        barrier = pltpu.get_barrier_semaphore()
        pl.semaphore_signal(barrier, inc=1, device_id=nbr_x,
                            device_id_type=pl.DeviceIdType.MESH)
        pl.semaphore_signal(barrier, inc=1, device_id=nbr_y,
                            device_id_type=pl.DeviceIdType.MESH)
        pl.semaphore_wait(barrier, 2)

        a1a = remote(x_ref.at[0, 0, pl.ds(p1A_first, E)], vrA.at[0],
                     sendA.at[P1A], recvA.at[P1A], nbr_x)
        a1b = remote(x_ref.at[0, 0, pl.ds(p1A_second, E)], vrA.at[1],
                     sendA.at[P1B], recvA.at[P1B], nbr_x)
        b1a = remote(x_ref.at[0, 0, pl.ds(p1B_first, E)], vrB.at[0],
                     sendB.at[P1A], recvB.at[P1A], nbr_y)
        b1b = remote(x_ref.at[0, 0, pl.ds(p1B_second, E)], vrB.at[1],
                     sendB.at[P1B], recvB.at[P1B], nbr_y)
        a1a.start()
        b1a.start()
        a1b.start()
        b1b.start()

        a1a.wait()
        pfs[0].wait()
        vrA[0] = vrA[0] + vxA_s[...]
        a2 = remote(vrA.at[0], r2A, sendA.at[P2], recvA.at[P2], nbr_y)
        a2.start()

        b1a.wait()
        pfs[1].wait()
        vrB[0] = vrB[0] + vxB_s[...]
        b2 = remote(vrB.at[0], r2B, sendB.at[P2], recvB.at[P2], nbr_x)
        b2.start()

        a1b.wait()
        pfs[2].wait()
        vrA[1] = vrA[1] + vxA_k[...]
        b1b.wait()
        pfs[3].wait()
        vrB[1] = vrB[1] + vxB_k[...]

        a2.wait()
        vrA[1] = vrA[1] + r2A[...]
        a3 = remote(vrA.at[1], out_ref.at[pl.ds(eA_keep, E)],
                    sendA.at[P3], recvA.at[P3], nbr_y)
        a3.start()
        a4a = remote(vrA.at[1], out_ref.at[pl.ds(eA_keep, E)],
                     sendA.at[P4A], recvA.at[P4A], nbr_x)
        a4a.start()
        stA = pltpu.make_async_copy(vrA.at[1], out_ref.at[pl.ds(eA_keep, E)], cpa)
        stA.start()

        b2.wait()
        vrB[1] = vrB[1] + r2B[...]
        b3 = remote(vrB.at[1], out_ref.at[pl.ds(eB_keep, E)],
                    sendB.at[P3], recvB.at[P3], nbr_x)
        b3.start()
        b4a = remote(vrB.at[1], out_ref.at[pl.ds(eB_keep, E)],
                     sendB.at[P4A], recvB.at[P4A], nbr_y)
        b4a.start()
        stB = pltpu.make_async_copy(vrB.at[1], out_ref.at[pl.ds(eB_keep, E)], cpb)
        stB.start()

        a3.wait()
        a4b = remote(out_ref.at[pl.ds(eA_send, E)], out_ref.at[pl.ds(eA_send, E)],
                     sendA.at[P4B], recvA.at[P4B], nbr_x)
        a4b.start()
        b3.wait()
        b4b = remote(out_ref.at[pl.ds(eB_send, E)], out_ref.at[pl.ds(eB_send, E)],
                     sendB.at[P4B], recvB.at[P4B], nbr_y)
        b4b.start()

        stA.wait()
        stB.wait()
        a4a.wait()
        b4a.wait()
        a4b.wait()
        b4b.wait()

    out = pl.pallas_call(
        body,
        out_shape=jax.ShapeDtypeStruct((M, N), jnp.float32),
        in_specs=[pl.BlockSpec(memory_space=pltpu.MemorySpace.HBM)],
        out_specs=pl.BlockSpec(memory_space=pltpu.MemorySpace.HBM),
        scratch_shapes=[
            pltpu.VMEM((2, E, N), jnp.float32),
            pltpu.VMEM((2, E, N), jnp.float32),
            pltpu.VMEM((E, N), jnp.float32),
            pltpu.VMEM((E, N), jnp.float32),
            pltpu.VMEM((E, N), jnp.float32),
            pltpu.VMEM((E, N), jnp.float32),
            pltpu.VMEM((E, N), jnp.float32),
            pltpu.VMEM((E, N), jnp.float32),
            pltpu.SemaphoreType.DMA((6,)),
            pltpu.SemaphoreType.DMA((6,)),
            pltpu.SemaphoreType.DMA((6,)),
            pltpu.SemaphoreType.DMA((6,)),
            pltpu.SemaphoreType.DMA,
            pltpu.SemaphoreType.DMA,
            pltpu.SemaphoreType.DMA,
            pltpu.SemaphoreType.DMA,
        ],
        compiler_params=pltpu.CompilerParams(
            collective_id=0,
            vmem_limit_bytes=56 * 1024 * 1024,
        ),
    )(x)
    return out
